# baseline (speedup 1.0000x reference)
import functools

import jax
import jax.numpy as jnp
from jax import lax
from jax.experimental import pallas as pl
from jax.experimental.pallas import tpu as pltpu
from jax.experimental.pallas import tpu_sc as plsc

NUM_VARS = 26
VOCAB = 100000
D_MODEL = 32
NUM_PATCHES = 16
BATCH = 4096

NUM_CORES = 2
NUM_SUBCORES = 16
NUM_WORKERS = NUM_CORES * NUM_SUBCORES
BB = BATCH // NUM_WORKERS
ROWS_PER_BLK = 4
NBLK = BB // ROWS_PER_BLK
GROWS = ROWS_PER_BLK * NUM_VARS
NPW = BB * NUM_VARS
ROW_F = NUM_PATCHES * D_MODEL
BLK_F = GROWS * ROW_F
LANES = 16
TOTAL_OUT = BATCH * NUM_VARS * ROW_F


def _emb_body(idx_hbm, tab_hbm, out_hbm, gid, grows, rep, lsem, gsem, wsem):
    wid = lax.axis_index("s") * NUM_CORES + lax.axis_index("c")
    e0 = wid * NPW

    pltpu.async_copy(idx_hbm.at[pl.ds(e0, NPW)], gid, lsem).wait()

    lane = lax.broadcasted_iota(jnp.int32, (LANES,), 0)

    def offset_step(i, _):
        sl = pl.ds(i * LANES, LANES)
        v = lax.rem(i * LANES + lane, NUM_VARS)
        gid[sl] = gid[sl] + v * VOCAB
        return 0

    lax.fori_loop(0, NPW // LANES, offset_step, 0)

    def gather(blk, buf):
        return pltpu.async_copy(
            tab_hbm.at[gid.at[pl.ds(blk * GROWS, GROWS)]], grows.at[buf], gsem
        )

    def replicate(buf):
        def rep_step(r, _):
            a = grows[buf, r, pl.ds(0, LANES)]
            b = grows[buf, r, pl.ds(LANES, LANES)]
            base = r * ROW_F
            for p in range(NUM_PATCHES):
                rep[buf, pl.ds(base + p * D_MODEL, LANES)] = a
                rep[buf, pl.ds(base + p * D_MODEL + LANES, LANES)] = b
            return 0

        lax.fori_loop(0, GROWS, rep_step, 0)

    def loop_body(j, _):
        g0 = gather(2 * j, 0)
        g1 = gather(2 * j + 1, 1)
        for t, g in ((0, g0), (1, g1)):
            g.wait()

            @pl.when(j >= 1)
            def _():
                pltpu.make_async_copy(
                    rep.at[t], out_hbm.at[pl.ds(0, BLK_F)], wsem
                ).wait()

            replicate(t)
            blk = 2 * j + t
            out0 = (e0 + blk * GROWS) * ROW_F
            pltpu.async_copy(rep.at[t], out_hbm.at[pl.ds(out0, BLK_F)], wsem)
        return 0

    lax.fori_loop(0, NBLK // 2, loop_body, 0)
    for _ in range(2):
        pltpu.make_async_copy(
            rep.at[0], out_hbm.at[pl.ds(0, BLK_F)], wsem
        ).wait()


@jax.jit
def _emb_call(idx_flat, tab_flat):
    mesh = plsc.VectorSubcoreMesh(core_axis_name="c", subcore_axis_name="s")
    return pl.kernel(
        _emb_body,
        out_type=jax.ShapeDtypeStruct((TOTAL_OUT,), jnp.float32),
        mesh=mesh,
        compiler_params=pltpu.CompilerParams(use_tc_tiling_on_sc=False),
        scratch_types=[
            pltpu.VMEM((NPW,), jnp.int32),
            pltpu.VMEM((2, GROWS, D_MODEL), jnp.float32),
            pltpu.VMEM((2, BLK_F), jnp.float32),
            pltpu.SemaphoreType.DMA,
            pltpu.SemaphoreType.DMA,
            pltpu.SemaphoreType.DMA,
        ],
    )(idx_flat, tab_flat)


def kernel(static_categorical_values, tables):
    idx_flat = static_categorical_values.astype(jnp.int32).reshape(-1)
    tab_flat = tables.reshape(NUM_VARS * VOCAB, D_MODEL)
    out = _emb_call(idx_flat, tab_flat)
    return out.reshape(BATCH, NUM_VARS, NUM_PATCHES, D_MODEL)


# speedup vs baseline: 1.6076x; 1.6076x over previous
import functools

import jax
import jax.numpy as jnp
from jax import lax
from jax.experimental import pallas as pl
from jax.experimental.pallas import tpu as pltpu
from jax.experimental.pallas import tpu_sc as plsc

NUM_VARS = 26
VOCAB = 100000
D_MODEL = 32
NUM_PATCHES = 16
BATCH = 4096

NUM_CORES = 2
NUM_SUBCORES = 16
NUM_WORKERS = NUM_CORES * NUM_SUBCORES
BB = 128
LANES = 16
NGRP = BB // LANES
PACK = 128 // D_MODEL
QROWS = NUM_VARS * VOCAB // PACK


def _emb_body(idx_hbm, tab_hbm, out_hbm, idxt, qbufall, qoffall, grows, tcol,
              isem, gsem, wsem):
    wid = lax.axis_index("s") * NUM_CORES + lax.axis_index("c")
    b0 = wid * BB
    lane = lax.broadcasted_iota(jnp.int32, (LANES,), 0)

    def gather_start(v, t):
        return pltpu.async_copy(
            tab_hbm.at[qbufall.at[lax.rem(v, 8)]], grows.at[t], gsem
        )

    def gather_wait(t):
        pltpu.make_async_copy(
            tab_hbm.at[qbufall.at[0]], grows.at[t], gsem
        ).wait()

    def drain_writes():
        for _ in range(NUM_PATCHES):
            pltpu.make_async_copy(
                tcol.at[0], out_hbm.at[0, 0, :, pl.ds(0, BB)], wsem
            ).wait()

    def extract(v, t):
        base = lax.rem(v, 8) * BB
        for d in range(D_MODEL):
            for g in range(NGRP):
                offv = qoffall[pl.ds(base + g * LANES, LANES)]
                tcol[t, d, pl.ds(g * LANES, LANES)] = plsc.load_gather(
                    grows.at[t], [lane + g * LANES, offv + d]
                )

    def body(j, _):
        for t in (0, 1):
            v = 2 * j + t
            if t == 0:
                @pl.when(lax.rem(v, 8) == 0)
                def _():
                    pltpu.async_copy(
                        idx_hbm.at[
                            pl.ds(pl.multiple_of(v, 8), 8), pl.ds(b0, BB)
                        ],
                        idxt,
                        isem,
                    ).wait()
                    for r in range(8):
                        for g in range(NGRP):
                            sl = pl.ds(g * LANES, LANES)
                            gidv = idxt[r, sl] + (v + r) * VOCAB
                            qbufall[r, sl] = lax.shift_right_logical(gidv, 2)
                            qoffall[pl.ds(r * BB + g * LANES, LANES)] = (
                                lax.shift_left(lax.bitwise_and(gidv, 3), 5)
                            )
                    gather_start(v, 0)

            gather_wait(t)

            @pl.when(
                jnp.logical_and(v + 1 < NUM_VARS, lax.rem(v + 1, 8) != 0)
            )
            def _():
                gather_start(v + 1, 1 - t)

            @pl.when(v >= 2)
            def _():
                drain_writes()

            extract(v, t)
            for p in range(NUM_PATCHES):
                pltpu.async_copy(
                    tcol.at[t], out_hbm.at[v, p, :, pl.ds(b0, BB)], wsem
                )
        return 0

    lax.fori_loop(0, NUM_VARS // 2, body, 0)
    drain_writes()
    drain_writes()


@jax.jit
def _emb_call(idx_t, tab_q):
    mesh = plsc.VectorSubcoreMesh(core_axis_name="c", subcore_axis_name="s")
    return pl.kernel(
        _emb_body,
        out_type=jax.ShapeDtypeStruct(
            (NUM_VARS, NUM_PATCHES, D_MODEL, BATCH), jnp.float32
        ),
        mesh=mesh,
        compiler_params=pltpu.CompilerParams(
            use_tc_tiling_on_sc=True, needs_layout_passes=False
        ),
        scratch_types=[
            pltpu.VMEM((8, BB), jnp.int32),
            pltpu.VMEM((8, BB), jnp.int32),
            pltpu.VMEM((8 * BB,), jnp.int32),
            pltpu.VMEM((2, BB, 128), jnp.float32),
            pltpu.VMEM((2, D_MODEL, BB), jnp.float32),
            pltpu.SemaphoreType.DMA,
            pltpu.SemaphoreType.DMA,
            pltpu.SemaphoreType.DMA,
        ],
    )(idx_t, tab_q)


def kernel(static_categorical_values, tables):
    idx_t = jnp.transpose(static_categorical_values.astype(jnp.int32))
    tab_q = tables.reshape(QROWS, 128)
    out = _emb_call(idx_t, tab_q)
    return jnp.transpose(out, (3, 0, 1, 2))
